# baseline (speedup 1.0000x reference)
import functools

import jax
import jax.numpy as jnp
from jax import lax
from jax.experimental import pallas as pl
from jax.experimental.pallas import tpu as pltpu
from jax.experimental.pallas import tpu_sc as plsc

NC = 2
NS = 16
NW = NC * NS

NB = 2
TCHUNK = 4096


def _transpose_body(x_ref, o_ref):
    o_ref[:, : x_ref.shape[0]] = x_ref[...].T


@functools.lru_cache(maxsize=None)
def _make_transpose(dim, vocab):
    grid = (vocab + TCHUNK - 1) // TCHUNK
    return pl.pallas_call(
        _transpose_body,
        grid=(grid,),
        in_specs=[pl.BlockSpec((dim, TCHUNK), lambda i: (0, i))],
        out_specs=pl.BlockSpec((TCHUNK, 2 * dim), lambda i: (i, 0)),
        out_shape=jax.ShapeDtypeStruct((vocab, 2 * dim), jnp.float32),
    )


@functools.lru_cache(maxsize=None)
def _make_gather(bsz, seq, dim):
    total = bsz * seq
    rows_w = bsz // NW
    chunks = rows_w // NB
    tok_c = NB * seq
    pieces = []
    off = 0
    while off < tok_c:
        n = min(128, tok_c - off)
        if (tok_c - off) > 128:
            n -= n % 8
        pieces.append((off, n))
        off += n
    assert all(o % 8 == 0 and n % 8 == 0 for o, n in pieces)
    assert chunks % 2 == 0 and seq % 8 == 0
    mesh = plsc.VectorSubcoreMesh(core_axis_name="c", subcore_axis_name="s")

    @functools.partial(
        pl.kernel,
        mesh=mesh,
        out_type=jax.ShapeDtypeStruct((total, 2 * dim), jnp.float32),
        scratch_types=[
            pltpu.VMEM((2, tok_c), jnp.int32),
            pltpu.VMEM((2, tok_c, 2 * dim), jnp.float32),
            pltpu.SemaphoreType.DMA,
            pltpu.SemaphoreType.DMA,
            pltpu.SemaphoreType.DMA,
            pltpu.SemaphoreType.DMA,
        ],
        compiler_params=pltpu.CompilerParams(use_tc_tiling_on_sc=False),
    )
    def grab(table_hbm, text_hbm, out_hbm, idx_v, rows_v, g0, g1, w0, w1):
        wid = lax.axis_index("s") * NC + lax.axis_index("c")
        gsems = (g0, g1)
        wsems = (w0, w1)

        def gather_list(b):
            return [
                (table_hbm.at[idx_v.at[b, pl.ds(off, n)]],
                 rows_v.at[b, pl.ds(off, n), :])
                for off, n in pieces
            ]

        def fire_chunk(g, b):
            row0 = wid * rows_w + g * NB
            for r in range(NB):
                pltpu.sync_copy(text_hbm.at[row0 + r, :],
                                idx_v.at[b, pl.ds(r * seq, seq)])
            for src, dst in gather_list(b):
                pltpu.async_copy(src, dst, gsems[b])

        def drain_gathers(b):
            for src, dst in gather_list(b):
                pltpu.make_async_copy(src, dst, gsems[b]).wait()

        def write_pair(g, b):
            base = (wid * rows_w + g * NB) * seq
            return (rows_v.at[b, :, pl.ds(0, dim)],
                    out_hbm.at[pl.ds(base, tok_c), pl.ds(0, dim)])

        def body(outer, carry):
            gA = 2 * outer

            for b in range(2):
                @pl.when(outer > 0)
                def _():
                    src, dst = write_pair(gA + b, b)
                    pltpu.make_async_copy(src, dst, wsems[b]).wait()
                fire_chunk(gA + b, b)

            for b in range(2):
                drain_gathers(b)
                src, dst = write_pair(gA + b, b)
                pltpu.async_copy(src, dst, wsems[b])

            return carry

        lax.fori_loop(0, chunks // 2, body, 0)
        last = chunks - 2
        for b in range(2):
            src, dst = write_pair(last + b, b)
            pltpu.make_async_copy(src, dst, wsems[b]).wait()

    return grab


def kernel(text, embedding_weight):
    b, l = text.shape
    vocab, dim = embedding_weight.shape
    tab128 = _make_transpose(dim, vocab)(embedding_weight.T)
    out = _make_gather(b, l, dim)(tab128, text)
    return out[:, :dim].reshape(b, l, dim)


# speedup vs baseline: 1.1428x; 1.1428x over previous
import functools

import jax
import jax.numpy as jnp
from jax import lax
from jax.experimental import pallas as pl
from jax.experimental.pallas import tpu as pltpu
from jax.experimental.pallas import tpu_sc as plsc

NC = 2
NS = 16
NW = NC * NS

NB = 2
TCHUNK = 16384


def _transpose_body(x_ref, o_ref):
    o_ref[:, : x_ref.shape[0]] = x_ref[...].T


@functools.lru_cache(maxsize=None)
def _make_transpose(dim, vocab):
    grid = (vocab + TCHUNK - 1) // TCHUNK
    return pl.pallas_call(
        _transpose_body,
        grid=(grid,),
        in_specs=[pl.BlockSpec((dim, TCHUNK), lambda i: (0, i))],
        out_specs=pl.BlockSpec((TCHUNK, 2 * dim), lambda i: (i, 0)),
        out_shape=jax.ShapeDtypeStruct((vocab, 2 * dim), jnp.float32),
    )


@functools.lru_cache(maxsize=None)
def _make_gather(bsz, seq, dim):
    total = bsz * seq
    rows_w = bsz // NW
    chunks = rows_w // NB
    tok_c = NB * seq
    pieces = []
    off = 0
    while off < tok_c:
        n = min(128, tok_c - off)
        if (tok_c - off) > 128:
            n -= n % 8
        pieces.append((off, n))
        off += n
    assert all(o % 8 == 0 and n % 8 == 0 for o, n in pieces)
    assert chunks % 2 == 0 and seq % 8 == 0
    mesh = plsc.VectorSubcoreMesh(core_axis_name="c", subcore_axis_name="s")

    @functools.partial(
        pl.kernel,
        mesh=mesh,
        out_type=jax.ShapeDtypeStruct((total, 2 * dim), jnp.float32),
        scratch_types=[
            pltpu.VMEM((2, tok_c), jnp.int32),
            pltpu.VMEM((2, tok_c, 2 * dim), jnp.float32),
            pltpu.SemaphoreType.DMA,
            pltpu.SemaphoreType.DMA,
            pltpu.SemaphoreType.DMA,
            pltpu.SemaphoreType.DMA,
        ],
        compiler_params=pltpu.CompilerParams(use_tc_tiling_on_sc=False),
    )
    def grab(table_hbm, text_hbm, out_hbm, idx_v, rows_v, g0, g1, w0, w1):
        wid = lax.axis_index("s") * NC + lax.axis_index("c")
        gsems = (g0, g1)
        wsems = (w0, w1)

        def gather_list(b):
            return [
                (table_hbm.at[idx_v.at[b, pl.ds(off, n)]],
                 rows_v.at[b, pl.ds(off, n), :])
                for off, n in pieces
            ]

        def fire_chunk(g, b):
            row0 = wid * rows_w + g * NB
            for r in range(NB):
                pltpu.sync_copy(text_hbm.at[row0 + r, :],
                                idx_v.at[b, pl.ds(r * seq, seq)])
            for src, dst in gather_list(b):
                pltpu.async_copy(src, dst, gsems[b])

        def drain_gathers(b):
            for src, dst in gather_list(b):
                pltpu.make_async_copy(src, dst, gsems[b]).wait()

        def write_pair(g, b):
            base = (wid * rows_w + g * NB) * seq
            return (rows_v.at[b], out_hbm.at[pl.ds(base, tok_c), :])

        def body(outer, carry):
            gA = 2 * outer

            for b in range(2):
                @pl.when(outer > 0)
                def _():
                    src, dst = write_pair(gA + b, b)
                    pltpu.make_async_copy(src, dst, wsems[b]).wait()
                fire_chunk(gA + b, b)

            for b in range(2):
                drain_gathers(b)
                src, dst = write_pair(gA + b, b)
                pltpu.async_copy(src, dst, wsems[b])

            return carry

        lax.fori_loop(0, chunks // 2, body, 0)
        last = chunks - 2
        for b in range(2):
            src, dst = write_pair(last + b, b)
            pltpu.make_async_copy(src, dst, wsems[b]).wait()

    return grab


def kernel(text, embedding_weight):
    b, l = text.shape
    vocab, dim = embedding_weight.shape
    tab128 = _make_transpose(dim, vocab)(embedding_weight.T)
    out = _make_gather(b, l, dim)(tab128, text)
    return out[:, :dim].reshape(b, l, dim)


# speedup vs baseline: 1.1514x; 1.0075x over previous
import functools

import jax
import jax.numpy as jnp
from jax import lax
from jax.experimental import pallas as pl
from jax.experimental.pallas import tpu as pltpu
from jax.experimental.pallas import tpu_sc as plsc

NC = 2
NS = 16
NW = NC * NS

NB = 2
TCHUNK = 32768


def _transpose_body(x_ref, o_ref):
    o_ref[:, : x_ref.shape[0]] = x_ref[...].T


@functools.lru_cache(maxsize=None)
def _make_transpose(dim, vocab):
    grid = (vocab + TCHUNK - 1) // TCHUNK
    return pl.pallas_call(
        _transpose_body,
        grid=(grid,),
        in_specs=[pl.BlockSpec((dim, TCHUNK), lambda i: (0, i))],
        out_specs=pl.BlockSpec((TCHUNK, 2 * dim), lambda i: (i, 0)),
        out_shape=jax.ShapeDtypeStruct((vocab, 2 * dim), jnp.float32),
    )


@functools.lru_cache(maxsize=None)
def _make_gather(bsz, seq, dim):
    total = bsz * seq
    rows_w = bsz // NW
    chunks = rows_w // NB
    tok_c = NB * seq
    pieces = []
    off = 0
    while off < tok_c:
        n = min(128, tok_c - off)
        if (tok_c - off) > 128:
            n -= n % 8
        pieces.append((off, n))
        off += n
    assert all(o % 8 == 0 and n % 8 == 0 for o, n in pieces)
    assert chunks % 2 == 0 and seq % 8 == 0
    mesh = plsc.VectorSubcoreMesh(core_axis_name="c", subcore_axis_name="s")

    @functools.partial(
        pl.kernel,
        mesh=mesh,
        out_type=jax.ShapeDtypeStruct((total, 2 * dim), jnp.float32),
        scratch_types=[
            pltpu.VMEM((2, tok_c), jnp.int32),
            pltpu.VMEM((2, tok_c, 2 * dim), jnp.float32),
            pltpu.SemaphoreType.DMA,
            pltpu.SemaphoreType.DMA,
            pltpu.SemaphoreType.DMA,
            pltpu.SemaphoreType.DMA,
        ],
        compiler_params=pltpu.CompilerParams(use_tc_tiling_on_sc=False),
    )
    def grab(table_hbm, text_hbm, out_hbm, idx_v, rows_v, g0, g1, w0, w1):
        wid = lax.axis_index("s") * NC + lax.axis_index("c")
        gsems = (g0, g1)
        wsems = (w0, w1)

        def gather_list(b):
            return [
                (table_hbm.at[idx_v.at[b, pl.ds(off, n)]],
                 rows_v.at[b, pl.ds(off, n), :])
                for off, n in pieces
            ]

        def fire_chunk(g, b):
            row0 = wid * rows_w + g * NB
            for r in range(NB):
                pltpu.sync_copy(text_hbm.at[row0 + r, :],
                                idx_v.at[b, pl.ds(r * seq, seq)])
            for src, dst in gather_list(b):
                pltpu.async_copy(src, dst, gsems[b])

        def drain_gathers(b):
            for src, dst in gather_list(b):
                pltpu.make_async_copy(src, dst, gsems[b]).wait()

        def write_pair(g, b):
            base = (wid * rows_w + g * NB) * seq
            return (rows_v.at[b], out_hbm.at[pl.ds(base, tok_c), :])

        def body(outer, carry):
            gA = 2 * outer

            for b in range(2):
                @pl.when(outer > 0)
                def _():
                    src, dst = write_pair(gA + b, b)
                    pltpu.make_async_copy(src, dst, wsems[b]).wait()
                fire_chunk(gA + b, b)

            for b in range(2):
                drain_gathers(b)
                src, dst = write_pair(gA + b, b)
                pltpu.async_copy(src, dst, wsems[b])

            return carry

        lax.fori_loop(0, chunks // 2, body, 0)
        last = chunks - 2
        for b in range(2):
            src, dst = write_pair(last + b, b)
            pltpu.make_async_copy(src, dst, wsems[b]).wait()

    return grab


def kernel(text, embedding_weight):
    b, l = text.shape
    vocab, dim = embedding_weight.shape
    tab128 = _make_transpose(dim, vocab)(embedding_weight.T)
    out = _make_gather(b, l, dim)(tab128, text)
    return out[:, :dim].reshape(b, l, dim)
